# baseline (speedup 1.0000x reference)
import functools

import jax
from jax import lax
from jax.experimental import pallas as pl
from jax.experimental.pallas import tpu as pltpu
from jax.experimental.pallas import tpu_sc as plsc


def _make_bcast_kernel(bsz, rows, dim, dtype):
    info = plsc.get_sparse_core_info()
    nc, ns = info.num_cores, info.num_subcores
    nw = nc * ns

    nt = dim // 128
    chunk = 64
    rpw = -(-rows // nw)
    assert rpw % chunk == 0
    nchunks = rpw // chunk
    tail = rows - (nw - 1) * rpw - (nchunks - 1) * chunk

    mesh = plsc.VectorSubcoreMesh(core_axis_name="c", subcore_axis_name="s")

    @functools.partial(
        pl.kernel,
        out_type=jax.ShapeDtypeStruct((rows, bsz * nt, 128), dtype),
        mesh=mesh,
        scratch_types=[
            pltpu.VMEM((chunk, dim), dtype),
            pltpu.VMEM((tail, dim), dtype),
            pltpu.SemaphoreType.DMA,
        ],
    )
    def bcast(w_hbm, y_hbm, buf, tailbuf, out_sem):
        wid = lax.axis_index("s") * nc + lax.axis_index("c")
        base = wid * rpw

        def do_chunk(s, n, b_ref):
            pltpu.sync_copy(w_hbm.at[pl.ds(s, n), :], b_ref)
            for j in range(nt):
                for b in range(bsz):
                    pltpu.async_copy(
                        b_ref.at[:, pl.ds(j * 128, 128)],
                        y_hbm.at[pl.ds(s, n), bsz * j + b, :],
                        out_sem,
                    )
            for j in range(nt):
                for b in range(bsz):
                    pltpu.make_async_copy(
                        b_ref.at[:, pl.ds(j * 128, 128)],
                        y_hbm.at[pl.ds(s, n), bsz * j + b, :],
                        out_sem,
                    ).wait()

        for i in range(nchunks - 1):
            do_chunk(pl.multiple_of(base + i * chunk, chunk), chunk, buf)

        s_last = pl.multiple_of(base + (nchunks - 1) * chunk, chunk)

        @pl.when(wid < nw - 1)
        def _():
            do_chunk(s_last, chunk, buf)

        @pl.when(wid == nw - 1)
        def _():
            do_chunk(rows - tail, tail, tailbuf)

    return bcast


def kernel(input, weight):
    bsz = input.shape[0]
    rows, dim = weight.shape
    nt = dim // 128
    fn = _make_bcast_kernel(bsz, rows, dim, weight.dtype)
    y = fn(weight)
    return y.reshape(rows, nt, bsz, 128).transpose(2, 0, 1, 3).reshape(bsz, rows, dim)


# speedup vs baseline: 1.0296x; 1.0296x over previous
import functools

import jax
from jax import lax
from jax.experimental import pallas as pl
from jax.experimental.pallas import tpu as pltpu
from jax.experimental.pallas import tpu_sc as plsc


def _make_bcast_kernel(bsz, rows, dim, dtype):
    info = plsc.get_sparse_core_info()
    nc, ns = info.num_cores, info.num_subcores
    nw = nc * ns

    nt = dim // 128
    chunk = 48
    rpw = -(-rows // nw)
    nfull = rpw // chunk
    rem = rpw - nfull * chunk
    tail = rem - (nw * rpw - rows)

    mesh = plsc.VectorSubcoreMesh(core_axis_name="c", subcore_axis_name="s")

    @functools.partial(
        pl.kernel,
        out_type=jax.ShapeDtypeStruct((rows, bsz * nt, 128), dtype),
        mesh=mesh,
        scratch_types=[
            pltpu.VMEM((chunk, dim), dtype),
            pltpu.VMEM((chunk, dim), dtype),
            pltpu.VMEM((tail, dim), dtype),
            pltpu.SemaphoreType.DMA,
            pltpu.SemaphoreType.DMA,
            pltpu.SemaphoreType.DMA,
        ],
    )
    def bcast(w_hbm, y_hbm, buf0, buf1, tailbuf, in0, in1, out_sem):
        wid = lax.axis_index("s") * nc + lax.axis_index("c")
        base = wid * rpw
        bufs = (buf0, buf1)
        in_sems = (in0, in1)

        def start_of(i):
            return pl.multiple_of(base + i * chunk, 16)

        def issue_writes(s, n, b_ref):
            for j in range(nt):
                for b in range(bsz):
                    pltpu.async_copy(
                        b_ref.at[:, pl.ds(j * 128, 128)],
                        y_hbm.at[pl.ds(s, n), bsz * j + b, :],
                        out_sem,
                    )

        def drain_writes(s, n, b_ref):
            for j in range(nt):
                for b in range(bsz):
                    pltpu.make_async_copy(
                        b_ref.at[:, pl.ds(j * 128, 128)],
                        y_hbm.at[pl.ds(s, n), bsz * j + b, :],
                        out_sem,
                    ).wait()

        pltpu.async_copy(w_hbm.at[pl.ds(start_of(0), chunk), :], bufs[0], in_sems[0])
        for i in range(nfull):
            s = start_of(i)
            pltpu.make_async_copy(
                w_hbm.at[pl.ds(s, chunk), :], bufs[i % 2], in_sems[i % 2]
            ).wait()
            if i >= 1:
                drain_writes(start_of(i - 1), chunk, bufs[(i - 1) % 2])
            if i + 1 < nfull:
                pltpu.async_copy(
                    w_hbm.at[pl.ds(start_of(i + 1), chunk), :],
                    bufs[(i + 1) % 2],
                    in_sems[(i + 1) % 2],
                )
            issue_writes(s, chunk, bufs[i % 2])
        drain_writes(start_of(nfull - 1), chunk, bufs[(nfull - 1) % 2])

        s_rem = start_of(nfull)
        rbuf = bufs[nfull % 2]

        @pl.when(wid < nw - 1)
        def _():
            rslice = rbuf.at[pl.ds(0, rem), :]
            pltpu.sync_copy(w_hbm.at[pl.ds(s_rem, rem), :], rslice)
            issue_writes(s_rem, rem, rslice)
            drain_writes(s_rem, rem, rslice)

        @pl.when(wid == nw - 1)
        def _():
            s_tail = rows - tail
            pltpu.sync_copy(w_hbm.at[pl.ds(s_tail, tail), :], tailbuf)
            issue_writes(s_tail, tail, tailbuf)
            drain_writes(s_tail, tail, tailbuf)

    return bcast


def kernel(input, weight):
    bsz = input.shape[0]
    rows, dim = weight.shape
    nt = dim // 128
    fn = _make_bcast_kernel(bsz, rows, dim, weight.dtype)
    y = fn(weight)
    return y.reshape(rows, nt, bsz, 128).transpose(2, 0, 1, 3).reshape(bsz, rows, dim)
